# baseline (speedup 1.0000x reference)
import functools

import jax
import jax.numpy as jnp
from jax import lax
from jax.experimental import pallas as pl
from jax.experimental.pallas import tpu as pltpu
from jax.experimental.pallas import tpu_sc as plsc

N = 10000
D = 128
E = 320000
E_TOT = E + N
NC = 2
NS = 16
NW = NC * NS
CHUNK = 128
CPW = 88
EPW = CPW * CHUNK
E_PAD = NW * EPW
N_PAD = 10240
STRIPE = N_PAD // NS

_mesh = plsc.VectorSubcoreMesh(core_axis_name="c", subcore_axis_name="s")


def _tc_pre_body(x_ref, w_ref, as_ref, ad_ref, h_ref, a2_ref, cm_ref):
    h = jnp.dot(x_ref[...], w_ref[...], preferred_element_type=jnp.float32)
    h_ref[...] = h
    a_src = jnp.sum(h * as_ref[...], axis=1)
    a_dst = jnp.sum(h * ad_ref[...], axis=1)
    a2_ref[...] = jnp.stack([a_src, a_dst])
    cm_ref[...] = jnp.maximum(jnp.max(a_src) + jnp.max(a_dst), 0.0).reshape(1, 1)


def _tc_pre(x, w, att_src, att_dst):
    return pl.pallas_call(
        _tc_pre_body,
        out_shape=(
            jax.ShapeDtypeStruct((N, D), jnp.float32),
            jax.ShapeDtypeStruct((2, N), jnp.float32),
            jax.ShapeDtypeStruct((1, 1), jnp.float32),
        ),
    )(x, w, att_src.reshape(1, D), att_dst.reshape(1, D))


def _tc_mid_body(p_ref, b_ref, w_ref, as_ref, ad_ref, h_ref, a2_ref, cm_ref):
    xin = p_ref[0] + p_ref[1] + b_ref[...]
    xin = jnp.where(xin >= 0, xin, 0.01 * xin)
    h = jnp.dot(xin, w_ref[...], preferred_element_type=jnp.float32)
    h_ref[...] = h
    a_src = jnp.sum(h * as_ref[...], axis=1)
    a_dst = jnp.sum(h * ad_ref[...], axis=1)
    a2_ref[...] = jnp.stack([a_src, a_dst])
    cm_ref[...] = jnp.maximum(jnp.max(a_src) + jnp.max(a_dst), 0.0).reshape(1, 1)


def _tc_mid(parts, b, w, att_src, att_dst):
    return pl.pallas_call(
        _tc_mid_body,
        out_shape=(
            jax.ShapeDtypeStruct((N, D), jnp.float32),
            jax.ShapeDtypeStruct((2, N), jnp.float32),
            jax.ShapeDtypeStruct((1, 1), jnp.float32),
        ),
    )(parts, b.reshape(1, D), w, att_src.reshape(1, D), att_dst.reshape(1, D))


def _tc_post_body(p_ref, b_ref, out_ref):
    s = p_ref[0] + p_ref[1] + b_ref[...]
    m = jnp.max(s, axis=0, keepdims=True)
    e = jnp.exp(s - m)
    out_ref[...] = e / jnp.sum(e, axis=0, keepdims=True)


def _tc_post(parts, b):
    return pl.pallas_call(
        _tc_post_body,
        out_shape=jax.ShapeDtypeStruct((N, D), jnp.float32),
    )(parts, b.reshape(1, D))


@functools.partial(
    pl.kernel,
    out_type=(
        jax.ShapeDtypeStruct((NW, CPW, CHUNK), jnp.float32),
        jax.ShapeDtypeStruct((NC, N_PAD), jnp.float32),
    ),
    mesh=_mesh,
    compiler_params=pltpu.CompilerParams(needs_layout_passes=False),
    scratch_types=[
        pltpu.VMEM((N,), jnp.float32),
        pltpu.VMEM((N,), jnp.float32),
        pltpu.VMEM((CPW, CHUNK), jnp.int32),
        pltpu.VMEM((CPW, CHUNK), jnp.int32),
        pltpu.VMEM((CPW, CHUNK), jnp.float32),
        pltpu.VMEM((STRIPE,), jnp.float32),
        pltpu.VMEM((16,), jnp.float32),
        pltpu.VMEM_SHARED((N_PAD,), jnp.float32),
    ],
)
def _sc_alpha(asrc_hbm, adst_hbm, srcp_hbm, dstp_hbm, c16_hbm,
              ex_hbm, den_hbm,
              asrc_v, adst_v, src_v, dst_v, ex_v, zero_v, c_v, den_s):
    c = lax.axis_index("c")
    s = lax.axis_index("s")
    wid = s * NC + c

    pltpu.sync_copy(asrc_hbm, asrc_v)
    pltpu.sync_copy(adst_hbm, adst_v)
    pltpu.sync_copy(srcp_hbm.at[wid], src_v)
    pltpu.sync_copy(dstp_hbm.at[wid], dst_v)
    pltpu.sync_copy(c16_hbm, c_v)

    def zero_body(i, carry):
        zero_v[pl.ds(i * 16, 16)] = jnp.zeros((16,), jnp.float32)
        return carry
    lax.fori_loop(0, STRIPE // 16, zero_body, 0)
    pltpu.sync_copy(zero_v, den_s.at[pl.ds(s * STRIPE, STRIPE)])

    cvec = c_v[...]
    base = wid * EPW
    lane = lax.iota(jnp.int32, 16)

    def alpha_body(j, carry):
        for k in range(CHUNK // 16):
            si = src_v[j, pl.ds(k * 16, 16)]
            di = dst_v[j, pl.ds(k * 16, 16)]
            a_s = plsc.load_gather(asrc_v, [si])
            a_d = plsc.load_gather(adst_v, [di])
            alpha = a_s + a_d
            alpha = jnp.where(alpha >= 0, alpha, 0.2 * alpha)
            ex = jnp.exp(alpha - cvec)
            eid = base + j * CHUNK + k * 16 + lane
            ex = jnp.where(eid < E_TOT, ex, 0.0)
            ex_v[j, pl.ds(k * 16, 16)] = ex
        return carry
    lax.fori_loop(0, CPW, alpha_body, 0)

    plsc.subcore_barrier()

    def scat_body(j, carry):
        pltpu.sync_copy(ex_v.at[j], den_s.at[dst_v.at[j]], add=True)
        return carry
    lax.fori_loop(0, CPW, scat_body, 0)

    plsc.subcore_barrier()

    pltpu.sync_copy(ex_v, ex_hbm.at[wid])

    @pl.when(s == 0)
    def _():
        pltpu.sync_copy(den_s, den_hbm.at[c])


@functools.partial(
    pl.kernel,
    out_type=jax.ShapeDtypeStruct((NW, CPW, CHUNK), jnp.float32),
    mesh=_mesh,
    compiler_params=pltpu.CompilerParams(needs_layout_passes=False),
    scratch_types=[
        pltpu.VMEM((N_PAD,), jnp.float32),
        pltpu.VMEM((N_PAD,), jnp.float32),
        pltpu.VMEM((CPW, CHUNK), jnp.int32),
        pltpu.VMEM((CPW, CHUNK), jnp.float32),
    ],
)
def _sc_coef(den_hbm, dstp_hbm, ex_hbm,
             cf_hbm,
             den0_v, den1_v, dst_v, cf_v):
    c = lax.axis_index("c")
    s = lax.axis_index("s")
    wid = s * NC + c

    pltpu.sync_copy(den_hbm.at[0], den0_v)
    pltpu.sync_copy(den_hbm.at[1], den1_v)
    pltpu.sync_copy(dstp_hbm.at[wid], dst_v)
    pltpu.sync_copy(ex_hbm.at[wid], cf_v)

    def coef_body(j, carry):
        for k in range(CHUNK // 16):
            sl = pl.ds(k * 16, 16)
            di = dst_v[j, sl]
            d = (plsc.load_gather(den0_v, [di])
                 + plsc.load_gather(den1_v, [di]) + 1e-16)
            cf_v[j, sl] = cf_v[j, sl] / d
        return carry
    lax.fori_loop(0, CPW, coef_body, 0)

    pltpu.sync_copy(cf_v, cf_hbm.at[wid])


GRP = 8


@functools.partial(
    pl.kernel,
    out_type=jax.ShapeDtypeStruct((NC, N, D), jnp.float32),
    mesh=_mesh,
    compiler_params=pltpu.CompilerParams(needs_layout_passes=False),
    scratch_types=[
        pltpu.VMEM((2 * GRP, CHUNK), jnp.int32),
        pltpu.VMEM((2 * GRP, CHUNK), jnp.int32),
        pltpu.VMEM((2 * GRP, CHUNK), jnp.float32),
        pltpu.VMEM((CHUNK, D), jnp.float32),
        pltpu.VMEM((CHUNK, D), jnp.float32),
        pltpu.VMEM_SHARED((N, D), jnp.float32),
        pltpu.SemaphoreType.DMA,
        pltpu.SemaphoreType.DMA,
    ],
)
def _sc_agg(h_hbm, srcp_hbm, dstp_hbm, cf_hbm,
            out_hbm,
            src_g, dst_g, cf_g, msg_a, msg_b, acc_s, sem_i, sem_g):
    c = lax.axis_index("c")
    s = lax.axis_index("s")
    wid = s * NC + c

    def grp_start(g):
        slot = (g % 2) * GRP
        hsl = pl.ds(g * GRP, GRP)
        vsl = pl.ds(slot, GRP)
        pltpu.async_copy(srcp_hbm.at[wid, hsl], src_g.at[vsl], sem_i)
        pltpu.async_copy(dstp_hbm.at[wid, hsl], dst_g.at[vsl], sem_i)
        pltpu.async_copy(cf_hbm.at[wid, hsl], cf_g.at[vsl], sem_i)

    def grp_wait(g):
        slot = (g % 2) * GRP
        hsl = pl.ds(g * GRP, GRP)
        vsl = pl.ds(slot, GRP)
        pltpu.make_async_copy(srcp_hbm.at[wid, hsl], src_g.at[vsl],
                              sem_i).wait()
        pltpu.make_async_copy(dstp_hbm.at[wid, hsl], dst_g.at[vsl],
                              sem_i).wait()
        pltpu.make_async_copy(cf_hbm.at[wid, hsl], cf_g.at[vsl],
                              sem_i).wait()

    def row_of(t):
        return ((t // GRP) % 2) * GRP + t % GRP

    def gat_start(t, msg):
        pltpu.async_copy(h_hbm.at[src_g.at[row_of(t)]], msg, sem_g)

    def gat_wait(t, msg):
        pltpu.make_async_copy(h_hbm.at[src_g.at[row_of(t)]], msg,
                              sem_g).wait()

    def zrow(i, carry):
        for k in range(D // 16):
            msg_a[i, pl.ds(k * 16, 16)] = jnp.zeros((16,), jnp.float32)
        return carry
    lax.fori_loop(0, CHUNK, zrow, 0)
    astripe = N // NS
    for i in range(astripe // CHUNK):
        pltpu.sync_copy(msg_a,
                        acc_s.at[pl.ds(s * astripe + i * CHUNK, CHUNK)])
    rem = astripe % CHUNK
    pltpu.sync_copy(msg_a.at[pl.ds(0, rem)],
                    acc_s.at[pl.ds(s * astripe + astripe - rem, rem)])

    plsc.subcore_barrier()

    def group_body(g, carry):
        hsl = pl.ds(g * GRP, GRP)
        vsl = pl.ds(0, GRP)
        pltpu.sync_copy(srcp_hbm.at[wid, hsl], src_g.at[vsl])

        def chunk(p, carry2):
            pltpu.async_copy(h_hbm.at[src_g.at[p]], msg_a, sem_g).wait()
            return carry2
        lax.fori_loop(0, GRP, chunk, 0)
        return carry
    lax.fori_loop(0, CPW // GRP, group_body, 0)

    plsc.subcore_barrier()

    @pl.when(s == 0)
    def _():
        pltpu.sync_copy(acc_s, out_hbm.at[c])


def _prep_edges(edge_index):
    loop = jnp.arange(N, dtype=edge_index.dtype)
    src = jnp.concatenate([edge_index[0], loop])
    dst = jnp.concatenate([edge_index[1], loop])
    src = jnp.pad(src, (0, E_PAD - E_TOT)).reshape(NW, CPW, CHUNK)
    dst = jnp.pad(dst, (0, E_PAD - E_TOT)).reshape(NW, CPW, CHUNK)
    return src, dst


def _gat_layer(h, a2, cm, srcp, dstp):
    c16 = jnp.full((16,), cm[0, 0], jnp.float32)
    ex, den = _sc_alpha(a2[0], a2[1], srcp, dstp, c16)
    cf = _sc_coef(den, dstp, ex)
    return _sc_agg(h, srcp, dstp, cf)


def kernel(x, edge_index, W1, att_src1, att_dst1, b1,
           W2, att_src2, att_dst2, b2):
    srcp, dstp = _prep_edges(edge_index)
    h1, a2_1, cm1 = _tc_pre(x, W1, att_src1, att_dst1)
    parts1 = _gat_layer(h1, a2_1, cm1, srcp, dstp)
    h2, a2_2, cm2 = _tc_mid(parts1, b1, W2, att_src2, att_dst2)
    parts2 = _gat_layer(h2, a2_2, cm2, srcp, dstp)
    return _tc_post(parts2, b2)


# speedup vs baseline: 3.1714x; 3.1714x over previous
import functools

import jax
import jax.numpy as jnp
from jax import lax
from jax.experimental import pallas as pl
from jax.experimental.pallas import tpu as pltpu
from jax.experimental.pallas import tpu_sc as plsc

N = 10000
D = 128
E = 320000
E_TOT = E + N
NC = 2
NS = 16
NW = NC * NS
CHUNK = 128
CPW = 81
EPW = CPW * CHUNK
E_PAD = NW * EPW
N_PAD = 10240
STRIPE = N_PAD // NS

_mesh = plsc.VectorSubcoreMesh(core_axis_name="c", subcore_axis_name="s")


def _tc_pre_body(x_ref, w_ref, as_ref, ad_ref, h_ref, a2_ref, cm_ref):
    h = jnp.dot(x_ref[...], w_ref[...], preferred_element_type=jnp.float32)
    h_ref[...] = h
    a_src = jnp.sum(h * as_ref[...], axis=1)
    a_dst = jnp.sum(h * ad_ref[...], axis=1)
    a2_ref[...] = jnp.stack([a_src, a_dst])
    cm_ref[...] = jnp.maximum(jnp.max(a_src) + jnp.max(a_dst), 0.0).reshape(1, 1)


def _tc_pre(x, w, att_src, att_dst):
    return pl.pallas_call(
        _tc_pre_body,
        out_shape=(
            jax.ShapeDtypeStruct((N, D), jnp.float32),
            jax.ShapeDtypeStruct((2, N), jnp.float32),
            jax.ShapeDtypeStruct((1, 1), jnp.float32),
        ),
    )(x, w, att_src.reshape(1, D), att_dst.reshape(1, D))


def _tc_mid_body(p_ref, b_ref, w_ref, as_ref, ad_ref, h_ref, a2_ref, cm_ref):
    xin = p_ref[0] + p_ref[1] + b_ref[...]
    xin = jnp.where(xin >= 0, xin, 0.01 * xin)
    h = jnp.dot(xin, w_ref[...], preferred_element_type=jnp.float32)
    h_ref[...] = h
    a_src = jnp.sum(h * as_ref[...], axis=1)
    a_dst = jnp.sum(h * ad_ref[...], axis=1)
    a2_ref[...] = jnp.stack([a_src, a_dst])
    cm_ref[...] = jnp.maximum(jnp.max(a_src) + jnp.max(a_dst), 0.0).reshape(1, 1)


def _tc_mid(parts, b, w, att_src, att_dst):
    return pl.pallas_call(
        _tc_mid_body,
        out_shape=(
            jax.ShapeDtypeStruct((N, D), jnp.float32),
            jax.ShapeDtypeStruct((2, N), jnp.float32),
            jax.ShapeDtypeStruct((1, 1), jnp.float32),
        ),
    )(parts, b.reshape(1, D), w, att_src.reshape(1, D), att_dst.reshape(1, D))


def _tc_post_body(p_ref, b_ref, out_ref):
    s = p_ref[0] + p_ref[1] + b_ref[...]
    m = jnp.max(s, axis=0, keepdims=True)
    e = jnp.exp(s - m)
    out_ref[...] = e / jnp.sum(e, axis=0, keepdims=True)


def _tc_post(parts, b):
    return pl.pallas_call(
        _tc_post_body,
        out_shape=jax.ShapeDtypeStruct((N, D), jnp.float32),
    )(parts, b.reshape(1, D))


@functools.partial(
    pl.kernel,
    out_type=(
        jax.ShapeDtypeStruct((NW, CPW, CHUNK), jnp.float32),
        jax.ShapeDtypeStruct((NC, N_PAD), jnp.float32),
    ),
    mesh=_mesh,
    compiler_params=pltpu.CompilerParams(needs_layout_passes=False),
    scratch_types=[
        pltpu.VMEM((N,), jnp.float32),
        pltpu.VMEM((N,), jnp.float32),
        pltpu.VMEM((CPW, CHUNK), jnp.int32),
        pltpu.VMEM((CPW, CHUNK), jnp.int32),
        pltpu.VMEM((CPW, CHUNK), jnp.float32),
        pltpu.VMEM((STRIPE,), jnp.float32),
        pltpu.VMEM((16,), jnp.float32),
        pltpu.VMEM_SHARED((N_PAD,), jnp.float32),
    ],
)
def _sc_alpha(asrc_hbm, adst_hbm, srcp_hbm, dstp_hbm, c16_hbm,
              ex_hbm, den_hbm,
              asrc_v, adst_v, src_v, dst_v, ex_v, zero_v, c_v, den_s):
    c = lax.axis_index("c")
    s = lax.axis_index("s")
    wid = s * NC + c

    pltpu.sync_copy(asrc_hbm, asrc_v)
    pltpu.sync_copy(adst_hbm, adst_v)
    pltpu.sync_copy(srcp_hbm.at[wid], src_v)
    pltpu.sync_copy(dstp_hbm.at[wid], dst_v)
    pltpu.sync_copy(c16_hbm, c_v)

    def zero_body(i, carry):
        zero_v[pl.ds(i * 16, 16)] = jnp.zeros((16,), jnp.float32)
        return carry
    lax.fori_loop(0, STRIPE // 16, zero_body, 0)
    pltpu.sync_copy(zero_v, den_s.at[pl.ds(s * STRIPE, STRIPE)])

    cvec = c_v[...]
    base = wid * EPW
    lane = lax.iota(jnp.int32, 16)

    def alpha_body(j, carry):
        for k in range(CHUNK // 16):
            si = src_v[j, pl.ds(k * 16, 16)]
            di = dst_v[j, pl.ds(k * 16, 16)]
            a_s = plsc.load_gather(asrc_v, [si])
            a_d = plsc.load_gather(adst_v, [di])
            alpha = a_s + a_d
            alpha = jnp.where(alpha >= 0, alpha, 0.2 * alpha)
            ex = jnp.exp(alpha - cvec)
            eid = base + j * CHUNK + k * 16 + lane
            ex = jnp.where(eid < E_TOT, ex, 0.0)
            ex_v[j, pl.ds(k * 16, 16)] = ex
        return carry
    lax.fori_loop(0, CPW, alpha_body, 0)

    plsc.subcore_barrier()

    def scat_body(j, carry):
        pltpu.sync_copy(ex_v.at[j], den_s.at[dst_v.at[j]], add=True)
        return carry
    lax.fori_loop(0, CPW, scat_body, 0)

    plsc.subcore_barrier()

    pltpu.sync_copy(ex_v, ex_hbm.at[wid])

    @pl.when(s == 0)
    def _():
        pltpu.sync_copy(den_s, den_hbm.at[c])


@functools.partial(
    pl.kernel,
    out_type=jax.ShapeDtypeStruct((NW, CPW, CHUNK), jnp.float32),
    mesh=_mesh,
    compiler_params=pltpu.CompilerParams(needs_layout_passes=False),
    scratch_types=[
        pltpu.VMEM((N_PAD,), jnp.float32),
        pltpu.VMEM((N_PAD,), jnp.float32),
        pltpu.VMEM((CPW, CHUNK), jnp.int32),
        pltpu.VMEM((CPW, CHUNK), jnp.float32),
    ],
)
def _sc_coef(den_hbm, dstp_hbm, ex_hbm,
             cf_hbm,
             den0_v, den1_v, dst_v, cf_v):
    c = lax.axis_index("c")
    s = lax.axis_index("s")
    wid = s * NC + c

    pltpu.sync_copy(den_hbm.at[0], den0_v)
    pltpu.sync_copy(den_hbm.at[1], den1_v)
    pltpu.sync_copy(dstp_hbm.at[wid], dst_v)
    pltpu.sync_copy(ex_hbm.at[wid], cf_v)

    def coef_body(j, carry):
        for k in range(CHUNK // 16):
            sl = pl.ds(k * 16, 16)
            di = dst_v[j, sl]
            d = (plsc.load_gather(den0_v, [di])
                 + plsc.load_gather(den1_v, [di]) + 1e-16)
            cf_v[j, sl] = cf_v[j, sl] / d
        return carry
    lax.fori_loop(0, CPW, coef_body, 0)

    pltpu.sync_copy(cf_v, cf_hbm.at[wid])


@functools.partial(
    pl.kernel,
    out_type=jax.ShapeDtypeStruct((NC, N, D), jnp.float32),
    mesh=_mesh,
    compiler_params=pltpu.CompilerParams(needs_layout_passes=False),
    scratch_types=[
        pltpu.VMEM((CPW, CHUNK), jnp.int32),
        pltpu.VMEM((CPW, CHUNK), jnp.int32),
        pltpu.VMEM((CPW, CHUNK), jnp.float32),
        pltpu.VMEM((CHUNK, D), jnp.float32),
        pltpu.VMEM_SHARED((N, D), jnp.float32),
        pltpu.SemaphoreType.DMA,
    ],
)
def _sc_agg(h_hbm, srcp_hbm, dstp_hbm, cf_hbm,
            out_hbm,
            src_v, dst_v, coef_v, msg_v, acc_s, sem):
    c = lax.axis_index("c")
    s = lax.axis_index("s")
    wid = s * NC + c

    pltpu.sync_copy(srcp_hbm.at[wid], src_v)
    pltpu.sync_copy(dstp_hbm.at[wid], dst_v)
    pltpu.sync_copy(cf_hbm.at[wid], coef_v)

    def zrow(i, carry):
        for k in range(D // 16):
            msg_v[i, pl.ds(k * 16, 16)] = jnp.zeros((16,), jnp.float32)
        return carry
    lax.fori_loop(0, CHUNK, zrow, 0)
    astripe = N // NS
    for i in range(astripe // CHUNK):
        pltpu.sync_copy(msg_v,
                        acc_s.at[pl.ds(s * astripe + i * CHUNK, CHUNK)])
    rem = astripe % CHUNK
    pltpu.sync_copy(msg_v.at[pl.ds(0, rem)],
                    acc_s.at[pl.ds(s * astripe + astripe - rem, rem)])

    plsc.subcore_barrier()

    def chunk_body(j, carry):
        pltpu.async_copy(h_hbm.at[src_v.at[j]], msg_v, sem).wait()

        def scale_row(r, carry2):
            cf = plsc.load_gather(coef_v, [jnp.full((16,), j, jnp.int32),
                                           jnp.full((16,), r, jnp.int32)])
            for k in range(D // 16):
                msg_v[r, pl.ds(k * 16, 16)] = msg_v[r, pl.ds(k * 16, 16)] * cf
            return carry2
        lax.fori_loop(0, CHUNK, scale_row, 0)

        pltpu.sync_copy(msg_v, acc_s.at[dst_v.at[j]], add=True)
        return carry
    lax.fori_loop(0, CPW, chunk_body, 0)

    plsc.subcore_barrier()

    @pl.when(s == 0)
    def _():
        pltpu.sync_copy(acc_s, out_hbm.at[c])


def _prep_edges(edge_index):
    loop = jnp.arange(N, dtype=edge_index.dtype)
    src = jnp.concatenate([edge_index[0], loop])
    dst = jnp.concatenate([edge_index[1], loop])
    src = jnp.pad(src, (0, E_PAD - E_TOT)).reshape(NW, CPW, CHUNK)
    dst = jnp.pad(dst, (0, E_PAD - E_TOT)).reshape(NW, CPW, CHUNK)
    return src, dst


def _gat_layer(h, a2, cm, srcp, dstp):
    c16 = jnp.full((16,), cm[0, 0], jnp.float32)
    ex, den = _sc_alpha(a2[0], a2[1], srcp, dstp, c16)
    cf = _sc_coef(den, dstp, ex)
    return _sc_agg(h, srcp, dstp, cf)


def kernel(x, edge_index, W1, att_src1, att_dst1, b1,
           W2, att_src2, att_dst2, b2):
    srcp, dstp = _prep_edges(edge_index)
    h1, a2_1, cm1 = _tc_pre(x, W1, att_src1, att_dst1)
    parts1 = _gat_layer(h1, a2_1, cm1, srcp, dstp)
    h2, a2_2, cm2 = _tc_mid(parts1, b1, W2, att_src2, att_dst2)
    parts2 = _gat_layer(h2, a2_2, cm2, srcp, dstp)
    return _tc_post(parts2, b2)


# speedup vs baseline: 3.3915x; 1.0694x over previous
import functools

import jax
import jax.numpy as jnp
from jax import lax
from jax.experimental import pallas as pl
from jax.experimental.pallas import tpu as pltpu
from jax.experimental.pallas import tpu_sc as plsc

N = 10000
D = 128
E = 320000
E_TOT = E + N
NC = 2
NS = 16
NW = NC * NS
CHUNK = 128
CPW = 81
EPW = CPW * CHUNK
E_PAD = NW * EPW
N_PAD = 10240
STRIPE = N_PAD // NS

_mesh = plsc.VectorSubcoreMesh(core_axis_name="c", subcore_axis_name="s")


def _tc_pre_body(x_ref, w_ref, as_ref, ad_ref, h_ref, a2_ref, cm_ref):
    h = jnp.dot(x_ref[...], w_ref[...], preferred_element_type=jnp.float32)
    h_ref[...] = h
    a_src = jnp.sum(h * as_ref[...], axis=1)
    a_dst = jnp.sum(h * ad_ref[...], axis=1)
    a2_ref[...] = jnp.stack([a_src, a_dst])
    cm_ref[...] = jnp.maximum(jnp.max(a_src) + jnp.max(a_dst), 0.0).reshape(1, 1)


def _tc_pre(x, w, att_src, att_dst):
    return pl.pallas_call(
        _tc_pre_body,
        out_shape=(
            jax.ShapeDtypeStruct((N, D), jnp.float32),
            jax.ShapeDtypeStruct((2, N), jnp.float32),
            jax.ShapeDtypeStruct((1, 1), jnp.float32),
        ),
    )(x, w, att_src.reshape(1, D), att_dst.reshape(1, D))


def _tc_mid_body(p_ref, den_ref, b_ref, w_ref, as_ref, ad_ref,
                 h_ref, a2_ref, cm_ref):
    d = den_ref[0, :N] + den_ref[1, :N] + 1e-16
    xin = (p_ref[0] + p_ref[1]) * (1.0 / d)[:, None] + b_ref[...]
    xin = jnp.where(xin >= 0, xin, 0.01 * xin)
    h = jnp.dot(xin, w_ref[...], preferred_element_type=jnp.float32)
    h_ref[...] = h
    a_src = jnp.sum(h * as_ref[...], axis=1)
    a_dst = jnp.sum(h * ad_ref[...], axis=1)
    a2_ref[...] = jnp.stack([a_src, a_dst])
    cm_ref[...] = jnp.maximum(jnp.max(a_src) + jnp.max(a_dst), 0.0).reshape(1, 1)


def _tc_mid(parts, den, b, w, att_src, att_dst):
    return pl.pallas_call(
        _tc_mid_body,
        out_shape=(
            jax.ShapeDtypeStruct((N, D), jnp.float32),
            jax.ShapeDtypeStruct((2, N), jnp.float32),
            jax.ShapeDtypeStruct((1, 1), jnp.float32),
        ),
    )(parts, den, b.reshape(1, D), w,
      att_src.reshape(1, D), att_dst.reshape(1, D))


def _tc_post_body(p_ref, den_ref, b_ref, out_ref):
    d = den_ref[0, :N] + den_ref[1, :N] + 1e-16
    s = (p_ref[0] + p_ref[1]) * (1.0 / d)[:, None] + b_ref[...]
    m = jnp.max(s, axis=0, keepdims=True)
    e = jnp.exp(s - m)
    out_ref[...] = e / jnp.sum(e, axis=0, keepdims=True)


def _tc_post(parts, den, b):
    return pl.pallas_call(
        _tc_post_body,
        out_shape=jax.ShapeDtypeStruct((N, D), jnp.float32),
    )(parts, den, b.reshape(1, D))


@functools.partial(
    pl.kernel,
    out_type=(
        jax.ShapeDtypeStruct((NW, CPW, CHUNK), jnp.float32),
        jax.ShapeDtypeStruct((NC, N_PAD), jnp.float32),
    ),
    mesh=_mesh,
    compiler_params=pltpu.CompilerParams(needs_layout_passes=False),
    scratch_types=[
        pltpu.VMEM((N,), jnp.float32),
        pltpu.VMEM((N,), jnp.float32),
        pltpu.VMEM((CPW, CHUNK), jnp.int32),
        pltpu.VMEM((CPW, CHUNK), jnp.int32),
        pltpu.VMEM((CPW, CHUNK), jnp.float32),
        pltpu.VMEM((STRIPE,), jnp.float32),
        pltpu.VMEM((16,), jnp.float32),
        pltpu.VMEM_SHARED((N_PAD,), jnp.float32),
    ],
)
def _sc_alpha(asrc_hbm, adst_hbm, srcp_hbm, dstp_hbm, c16_hbm,
              ex_hbm, den_hbm,
              asrc_v, adst_v, src_v, dst_v, ex_v, zero_v, c_v, den_s):
    c = lax.axis_index("c")
    s = lax.axis_index("s")
    wid = s * NC + c

    pltpu.sync_copy(asrc_hbm, asrc_v)
    pltpu.sync_copy(adst_hbm, adst_v)
    pltpu.sync_copy(srcp_hbm.at[wid], src_v)
    pltpu.sync_copy(dstp_hbm.at[wid], dst_v)
    pltpu.sync_copy(c16_hbm, c_v)

    def zero_body(i, carry):
        zero_v[pl.ds(i * 16, 16)] = jnp.zeros((16,), jnp.float32)
        return carry
    lax.fori_loop(0, STRIPE // 16, zero_body, 0)
    pltpu.sync_copy(zero_v, den_s.at[pl.ds(s * STRIPE, STRIPE)])

    cvec = c_v[...]
    base = wid * EPW
    lane = lax.iota(jnp.int32, 16)

    def alpha_body(j, carry):
        for k in range(CHUNK // 16):
            si = src_v[j, pl.ds(k * 16, 16)]
            di = dst_v[j, pl.ds(k * 16, 16)]
            a_s = plsc.load_gather(asrc_v, [si])
            a_d = plsc.load_gather(adst_v, [di])
            alpha = a_s + a_d
            alpha = jnp.where(alpha >= 0, alpha, 0.2 * alpha)
            ex = jnp.exp(alpha - cvec)
            eid = base + j * CHUNK + k * 16 + lane
            ex = jnp.where(eid < E_TOT, ex, 0.0)
            ex_v[j, pl.ds(k * 16, 16)] = ex
        return carry
    lax.fori_loop(0, CPW, alpha_body, 0)

    plsc.subcore_barrier()

    def scat_body(j, carry):
        pltpu.sync_copy(ex_v.at[j], den_s.at[dst_v.at[j]], add=True)
        return carry
    lax.fori_loop(0, CPW, scat_body, 0)

    plsc.subcore_barrier()

    pltpu.sync_copy(ex_v, ex_hbm.at[wid])

    @pl.when(s == 0)
    def _():
        pltpu.sync_copy(den_s, den_hbm.at[c])


@functools.partial(
    pl.kernel,
    out_type=jax.ShapeDtypeStruct((NC, N, D), jnp.float32),
    mesh=_mesh,
    compiler_params=pltpu.CompilerParams(needs_layout_passes=False),
    scratch_types=[
        pltpu.VMEM((CPW, CHUNK), jnp.int32),
        pltpu.VMEM((CPW, CHUNK), jnp.int32),
        pltpu.VMEM((CPW, CHUNK), jnp.float32),
        pltpu.VMEM((CHUNK, D), jnp.float32),
        pltpu.VMEM_SHARED((N, D), jnp.float32),
        pltpu.SemaphoreType.DMA,
    ],
)
def _sc_agg(h_hbm, srcp_hbm, dstp_hbm, ex_hbm,
            out_hbm,
            src_v, dst_v, coef_v, msg_v, acc_s, sem):
    c = lax.axis_index("c")
    s = lax.axis_index("s")
    wid = s * NC + c

    pltpu.sync_copy(srcp_hbm.at[wid], src_v)
    pltpu.sync_copy(dstp_hbm.at[wid], dst_v)
    pltpu.sync_copy(ex_hbm.at[wid], coef_v)

    def zrow(i, carry):
        for k in range(D // 16):
            msg_v[i, pl.ds(k * 16, 16)] = jnp.zeros((16,), jnp.float32)
        return carry
    lax.fori_loop(0, CHUNK, zrow, 0)
    astripe = N // NS
    for i in range(astripe // CHUNK):
        pltpu.sync_copy(msg_v,
                        acc_s.at[pl.ds(s * astripe + i * CHUNK, CHUNK)])
    rem = astripe % CHUNK
    pltpu.sync_copy(msg_v.at[pl.ds(0, rem)],
                    acc_s.at[pl.ds(s * astripe + astripe - rem, rem)])

    plsc.subcore_barrier()

    def chunk_body(j, carry):
        pltpu.async_copy(h_hbm.at[src_v.at[j]], msg_v, sem).wait()

        def scale_row(r, carry2):
            cf = plsc.load_gather(coef_v, [jnp.full((16,), j, jnp.int32),
                                           jnp.full((16,), r, jnp.int32)])
            for k in range(D // 16):
                msg_v[r, pl.ds(k * 16, 16)] = msg_v[r, pl.ds(k * 16, 16)] * cf
            return carry2
        lax.fori_loop(0, CHUNK, scale_row, 0)

        pltpu.sync_copy(msg_v, acc_s.at[dst_v.at[j]], add=True)
        return carry
    lax.fori_loop(0, CPW, chunk_body, 0)

    plsc.subcore_barrier()

    @pl.when(s == 0)
    def _():
        pltpu.sync_copy(acc_s, out_hbm.at[c])


def _prep_edges(edge_index):
    loop = jnp.arange(N, dtype=edge_index.dtype)
    src = jnp.concatenate([edge_index[0], loop])
    dst = jnp.concatenate([edge_index[1], loop])
    src = jnp.pad(src, (0, E_PAD - E_TOT)).reshape(NW, CPW, CHUNK)
    dst = jnp.pad(dst, (0, E_PAD - E_TOT)).reshape(NW, CPW, CHUNK)
    return src, dst


def _gat_layer(h, a2, cm, srcp, dstp):
    c16 = jnp.full((16,), cm[0, 0], jnp.float32)
    ex, den = _sc_alpha(a2[0], a2[1], srcp, dstp, c16)
    return _sc_agg(h, srcp, dstp, ex), den


def kernel(x, edge_index, W1, att_src1, att_dst1, b1,
           W2, att_src2, att_dst2, b2):
    srcp, dstp = _prep_edges(edge_index)
    h1, a2_1, cm1 = _tc_pre(x, W1, att_src1, att_dst1)
    parts1, den1 = _gat_layer(h1, a2_1, cm1, srcp, dstp)
    h2, a2_2, cm2 = _tc_mid(parts1, den1, b1, W2, att_src2, att_dst2)
    parts2, den2 = _gat_layer(h2, a2_2, cm2, srcp, dstp)
    return _tc_post(parts2, den2, b2)
